# baseline (speedup 1.0000x reference)
import functools

import jax
import jax.numpy as jnp
from jax import lax
from jax.experimental import pallas as pl
from jax.experimental.pallas import tpu as pltpu
from jax.experimental.pallas import tpu_sc as plsc

E = 1048576
K = 20
B = 2048
NCOL = 2 * K
NACC = NCOL + 1
SUB = 128

NHALF = 8
EH = E // NHALF
ROWS_H = EH // SUB * NCOL

EC2 = 8192
NSTEP_H = EH // EC2
SC_PER_STEP = EC2 // SUB
ROWS_PER_STEP = SC_PER_STEP * NCOL

NC = 2
NS = 16
NW = NC * NS
EW = EH // NW
CH = 512
CHS = CH // SUB
NCH = EW // CH
NG = CH // 16


def _pack_kernel(label_ref, theta_ref, alpha_ref, idx_ref, idxn_ref,
                 out_ref):
    theta = theta_ref[...]
    alpha = alpha_ref[...]
    lab = label_ref[...].reshape(EC2, 1)
    m = (idx_ref[...] == idxn_ref[...]).astype(jnp.float32).reshape(EC2, 1)
    bce = (jnp.maximum(theta, 0.0) - theta * lab
           + jnp.log1p(jnp.exp(-jnp.abs(theta)))) * m
    pack = jnp.concatenate([bce, alpha], axis=1)
    x = pack.reshape(SC_PER_STEP, SUB, NCOL)
    ident = jnp.eye(SUB, dtype=jnp.float32)
    t = lax.dot_general(x, ident, (((1,), (0,)), ((), ())),
                        preferred_element_type=jnp.float32)
    out_ref[...] = t.reshape(ROWS_PER_STEP, SUB)


def _make_sc_body(half):
    def _sc_body(p_h, ixp_h, out_h, acc_v, pv, ix_v):
        cid = lax.axis_index("c")
        sid = lax.axis_index("s")
        wid = sid * NC + cid
        base = wid * EW

        zero16 = jnp.zeros((16,), jnp.float32)

        def zero_body(i, _):
            acc_v[pl.ds(i * 16, 16)] = zero16
            return _

        lax.fori_loop(0, NACC * B // 16, zero_body, None)

        iota = lax.iota(jnp.int32, 16)
        lane0 = iota == 0
        lane15 = iota == 15
        cnt_end = (iota + 1).astype(jnp.float32)
        cnt_start = iota.astype(jnp.float32)

        def chunk_body(ci, _):
            e0 = base + ci * CH
            row0 = (e0 // SUB) * NCOL
            pltpu.sync_copy(p_h.at[pl.ds(row0, CHS * NCOL)], pv)
            pltpu.sync_copy(ixp_h.at[pl.ds(half * EH + e0, CH + 16)], ix_v)

            @plsc.parallel_loop(0, NG)
            def group_body(g):
                o = g * 16
                s = g // 8
                l = g % 8
                d = ix_v[pl.ds(8 + o, 16)]
                dn = ix_v[pl.ds(9 + o, 16)]
                dp = ix_v[pl.ds(7 + o, 16)]
                m_end = (d != dn) | lane15
                m_start = (d != dp) | lane0
                srow = s * NCOL
                lo = l * 16

                for col in range(NCOL):
                    v = pv[srow + col, pl.ds(lo, 16)]
                    c_in = plsc.cumsum(v)
                    x_ex = v - c_in
                    dk = d + (col * B)
                    plsc.addupdate_scatter(acc_v, [dk], c_in, mask=m_end)
                    plsc.addupdate_scatter(acc_v, [dk], x_ex, mask=m_start)

                dc = d + (NCOL * B)
                plsc.addupdate_scatter(acc_v, [dc], cnt_end, mask=m_end)
                plsc.addupdate_scatter(acc_v, [dc], -cnt_start,
                                       mask=m_start)

            return _

        lax.fori_loop(0, NCH, chunk_body, None)
        pltpu.sync_copy(acc_v, out_h.at[wid])

    return _sc_body


def _tc_combine_kernel(p_ref, out_ref, acc_ref):
    step = pl.program_id(0)
    contrib = jnp.sum(p_ref[...], axis=(0, 1))

    @pl.when(step == 0)
    def _():
        acc_ref[...] = contrib

    @pl.when(step != 0)
    def _():
        acc_ref[...] += contrib

    @pl.when(step != NHALF - 1)
    def _():
        out_ref[...] = jnp.zeros((1, 1), jnp.float32)

    @pl.when(step == NHALF - 1)
    def _():
        _combine_epilogue(acc_ref, out_ref)


def _combine_epilogue(acc_ref, out_ref):
    S = acc_ref[...]
    nll = S[0:K]
    A = S[K:2 * K]
    n = S[2 * K:2 * K + 1]
    ra = A / n
    ra_max = jnp.max(ra, axis=0, keepdims=True)
    ls = ra - ra_max - jnp.log(
        jnp.sum(jnp.exp(ra - ra_max), axis=0, keepdims=True))
    x = -nll + ls
    x_max = jnp.max(x, axis=0, keepdims=True)
    lp = x_max + jnp.log(jnp.sum(jnp.exp(x - x_max), axis=0,
                                 keepdims=True))
    loss_b = -lp / n
    out_ref[...] = jnp.sum(loss_b, axis=1, keepdims=True) / B


def _prepass(half, label, log_theta, log_alpha, idx, idxn):
    off = half * (EH // EC2)
    return pl.pallas_call(
        _pack_kernel,
        grid=(NSTEP_H,),
        in_specs=[
            pl.BlockSpec((EC2,), lambda i: (i + off,)),
            pl.BlockSpec((EC2, K), lambda i: (i + off, 0)),
            pl.BlockSpec((EC2, K), lambda i: (i + off, 0)),
            pl.BlockSpec((EC2,), lambda i: (i + off,)),
            pl.BlockSpec((EC2,), lambda i: (i + off,)),
        ],
        out_specs=pl.BlockSpec((ROWS_PER_STEP, SUB), lambda i: (i, 0)),
        out_shape=jax.ShapeDtypeStruct((ROWS_H, SUB), jnp.float32),
    )(label, log_theta, log_alpha, idx, idxn)


def _sc_reduce(half, packed, ixp):
    mesh = plsc.VectorSubcoreMesh(core_axis_name="c", subcore_axis_name="s",
                                  num_cores=NC, num_subcores=NS)
    return pl.kernel(
        _make_sc_body(half),
        out_type=jax.ShapeDtypeStruct((NW, NACC * B), jnp.float32),
        mesh=mesh,
        compiler_params=pltpu.CompilerParams(needs_layout_passes=False),
        scratch_types=[
            pltpu.VMEM((NACC * B,), jnp.float32),
            pltpu.VMEM((CHS * NCOL, SUB), jnp.float32),
            pltpu.VMEM((CH + 16,), jnp.int32),
        ],
    )(packed, ixp)


@jax.jit
def _run(label, log_theta, log_alpha, subgraph_idx):
    idx = subgraph_idx.astype(jnp.int32)
    idxn = jnp.concatenate([idx[1:], jnp.full((1,), B, jnp.int32)])
    ixp = jnp.concatenate([jnp.full((8,), -1, jnp.int32), idx,
                           jnp.full((8,), B, jnp.int32)])

    parts = []
    for h in range(NHALF):
        p = _prepass(h, label, log_theta, log_alpha, idx, idxn)
        parts.append(_sc_reduce(h, p, ixp).reshape(1, NW, NACC, B))
    stacked = jnp.concatenate(parts, axis=0)

    out = pl.pallas_call(
        _tc_combine_kernel,
        grid=(NHALF,),
        in_specs=[pl.BlockSpec((1, NW, NACC, B), lambda i: (i, 0, 0, 0))],
        out_specs=pl.BlockSpec((1, 1), lambda i: (0, 0)),
        out_shape=jax.ShapeDtypeStruct((1, 1), jnp.float32),
        scratch_shapes=[pltpu.VMEM((NACC, B), jnp.float32)],
    )(stacked)
    return out[0, 0]


def kernel(label, log_theta, log_alpha, subgraph_idx, subgraph_idx_base,
           num_canonical_order):
    loss = _run(label, log_theta, log_alpha, subgraph_idx)
    return loss * jnp.asarray(num_canonical_order, jnp.float32)


# speedup vs baseline: 1.0770x; 1.0770x over previous
import functools

import jax
import jax.numpy as jnp
from jax import lax
from jax.experimental import pallas as pl
from jax.experimental.pallas import tpu as pltpu
from jax.experimental.pallas import tpu_sc as plsc

E = 1048576
K = 20
B = 2048
NCOL = 2 * K
NACC = NCOL + 1
SUB = 128

NHALF = 4
EH = E // NHALF
ROWS_H = EH // SUB * NCOL

EC2 = 8192
NSTEP_H = EH // EC2
SC_PER_STEP = EC2 // SUB
ROWS_PER_STEP = SC_PER_STEP * NCOL

NC = 2
NS = 16
NW = NC * NS
EW = EH // NW
CH = 512
CHS = CH // SUB
NCH = EW // CH
NG = CH // 16


def _pack_kernel(label_ref, theta_ref, alpha_ref, idx_ref, idxn_ref,
                 out_ref):
    theta = theta_ref[...]
    alpha = alpha_ref[...]
    lab = label_ref[...].reshape(EC2, 1)
    m = (idx_ref[...] == idxn_ref[...]).astype(jnp.float32).reshape(EC2, 1)
    bce = (jnp.maximum(theta, 0.0) - theta * lab
           + jnp.log1p(jnp.exp(-jnp.abs(theta)))) * m
    pack = jnp.concatenate([bce, alpha], axis=1)
    x = pack.reshape(SC_PER_STEP, SUB, NCOL)
    ident = jnp.eye(SUB, dtype=jnp.float32)
    t = lax.dot_general(x, ident, (((1,), (0,)), ((), ())),
                        preferred_element_type=jnp.float32)
    out_ref[...] = t


def _make_sc_body(half):
    def _sc_body(p_h, ixp_h, out_h, acc_v, pv, ix_v):
        cid = lax.axis_index("c")
        sid = lax.axis_index("s")
        wid = sid * NC + cid
        base = wid * EW

        zero16 = jnp.zeros((16,), jnp.float32)

        def zero_body(i, _):
            acc_v[pl.ds(i * 16, 16)] = zero16
            return _

        lax.fori_loop(0, NACC * B // 16, zero_body, None)

        iota = lax.iota(jnp.int32, 16)
        lane0 = iota == 0
        lane15 = iota == 15
        cnt_end = (iota + 1).astype(jnp.float32)
        cnt_start = iota.astype(jnp.float32)

        def chunk_body(ci, _):
            e0 = base + ci * CH
            sub0 = e0 // SUB
            pltpu.sync_copy(p_h.at[pl.ds(sub0, CHS)], pv)
            pltpu.sync_copy(ixp_h.at[pl.ds(half * EH + e0, CH + 16)], ix_v)

            @plsc.parallel_loop(0, NG)
            def group_body(g):
                o = g * 16
                s = g // 8
                l = g % 8
                d = ix_v[pl.ds(8 + o, 16)]
                dn = ix_v[pl.ds(9 + o, 16)]
                dp = ix_v[pl.ds(7 + o, 16)]
                m_end = (d != dn) | lane15
                m_start = (d != dp) | lane0
                lo = l * 16

                for col in range(NCOL):
                    v = pv[s, col, pl.ds(lo, 16)]
                    c_in = plsc.cumsum(v)
                    x_ex = v - c_in
                    dk = d + (col * B)
                    plsc.addupdate_scatter(acc_v, [dk], c_in, mask=m_end)
                    plsc.addupdate_scatter(acc_v, [dk], x_ex, mask=m_start)

                dc = d + (NCOL * B)
                plsc.addupdate_scatter(acc_v, [dc], cnt_end, mask=m_end)
                plsc.addupdate_scatter(acc_v, [dc], -cnt_start,
                                       mask=m_start)

            return _

        lax.fori_loop(0, NCH, chunk_body, None)
        pltpu.sync_copy(acc_v, out_h.at[wid])

    return _sc_body


def _tc_combine_kernel(p_ref, out_ref, acc_ref):
    step = pl.program_id(0)
    contrib = jnp.sum(p_ref[...], axis=(0, 1))

    @pl.when(step == 0)
    def _():
        acc_ref[...] = contrib

    @pl.when(step != 0)
    def _():
        acc_ref[...] += contrib

    @pl.when(step != NHALF - 1)
    def _():
        out_ref[...] = jnp.zeros((1, 1), jnp.float32)

    @pl.when(step == NHALF - 1)
    def _():
        _combine_epilogue(acc_ref, out_ref)


def _combine_epilogue(acc_ref, out_ref):
    S = acc_ref[...]
    nll = S[0:K]
    A = S[K:2 * K]
    n = S[2 * K:2 * K + 1]
    ra = A / n
    ra_max = jnp.max(ra, axis=0, keepdims=True)
    ls = ra - ra_max - jnp.log(
        jnp.sum(jnp.exp(ra - ra_max), axis=0, keepdims=True))
    x = -nll + ls
    x_max = jnp.max(x, axis=0, keepdims=True)
    lp = x_max + jnp.log(jnp.sum(jnp.exp(x - x_max), axis=0,
                                 keepdims=True))
    loss_b = -lp / n
    out_ref[...] = jnp.sum(loss_b, axis=1, keepdims=True) / B


def _prepass(half, label, log_theta, log_alpha, idx, idxn):
    off = half * (EH // EC2)
    return pl.pallas_call(
        _pack_kernel,
        grid=(NSTEP_H,),
        in_specs=[
            pl.BlockSpec((EC2,), lambda i: (i + off,)),
            pl.BlockSpec((EC2, K), lambda i: (i + off, 0)),
            pl.BlockSpec((EC2, K), lambda i: (i + off, 0)),
            pl.BlockSpec((EC2,), lambda i: (i + off,)),
            pl.BlockSpec((EC2,), lambda i: (i + off,)),
        ],
        out_specs=pl.BlockSpec((SC_PER_STEP, NCOL, SUB),
                               lambda i: (i, 0, 0)),
        out_shape=jax.ShapeDtypeStruct((EH // SUB, NCOL, SUB),
                                       jnp.float32),
    )(label, log_theta, log_alpha, idx, idxn)


def _sc_reduce(half, packed, ixp):
    mesh = plsc.VectorSubcoreMesh(core_axis_name="c", subcore_axis_name="s",
                                  num_cores=NC, num_subcores=NS)
    return pl.kernel(
        _make_sc_body(half),
        out_type=jax.ShapeDtypeStruct((NW, NACC * B), jnp.float32),
        mesh=mesh,
        compiler_params=pltpu.CompilerParams(needs_layout_passes=False),
        scratch_types=[
            pltpu.VMEM((NACC * B,), jnp.float32),
            pltpu.VMEM((CHS, NCOL, SUB), jnp.float32),
            pltpu.VMEM((CH + 16,), jnp.int32),
        ],
    )(packed, ixp)


@jax.jit
def _run(label, log_theta, log_alpha, subgraph_idx):
    idx = subgraph_idx.astype(jnp.int32)
    idxn = jnp.concatenate([idx[1:], jnp.full((1,), B, jnp.int32)])
    ixp = jnp.concatenate([jnp.full((8,), -1, jnp.int32), idx,
                           jnp.full((8,), B, jnp.int32)])

    parts = []
    for h in range(NHALF):
        p = _prepass(h, label, log_theta, log_alpha, idx, idxn)
        parts.append(_sc_reduce(h, p, ixp).reshape(1, NW, NACC, B))
    stacked = jnp.concatenate(parts, axis=0)

    out = pl.pallas_call(
        _tc_combine_kernel,
        grid=(NHALF,),
        in_specs=[pl.BlockSpec((1, NW, NACC, B), lambda i: (i, 0, 0, 0))],
        out_specs=pl.BlockSpec((1, 1), lambda i: (0, 0)),
        out_shape=jax.ShapeDtypeStruct((1, 1), jnp.float32),
        scratch_shapes=[pltpu.VMEM((NACC, B), jnp.float32)],
    )(stacked)
    return out[0, 0]


def kernel(label, log_theta, log_alpha, subgraph_idx, subgraph_idx_base,
           num_canonical_order):
    loss = _run(label, log_theta, log_alpha, subgraph_idx)
    return loss * jnp.asarray(num_canonical_order, jnp.float32)


# speedup vs baseline: 1.0872x; 1.0095x over previous
import functools

import jax
import jax.numpy as jnp
from jax import lax
from jax.experimental import pallas as pl
from jax.experimental.pallas import tpu as pltpu
from jax.experimental.pallas import tpu_sc as plsc

E = 1048576
K = 20
B = 2048
NCOL = 2 * K
NACC = NCOL + 1
SUB = 128

NHALF = 4
EH = E // NHALF
ROWS_H = EH // SUB * NCOL

EC2 = 8192
NSTEP_H = EH // EC2
SC_PER_STEP = EC2 // SUB
ROWS_PER_STEP = SC_PER_STEP * NCOL

NC = 2
NS = 16
NW = NC * NS
EW = EH // NW
CH = 512
CHS = CH // SUB
NCH = EW // CH
NG = CH // 16


def _pack_kernel(label_ref, theta_ref, alpha_ref, idx_ref, idxn_ref,
                 out_ref):
    theta = theta_ref[...]
    alpha = alpha_ref[...]
    lab = label_ref[...].reshape(EC2, 1)
    m = (idx_ref[...] == idxn_ref[...]).astype(jnp.float32).reshape(EC2, 1)
    bce = (jnp.maximum(theta, 0.0) - theta * lab
           + jnp.log1p(jnp.exp(-jnp.abs(theta)))) * m
    pack = jnp.concatenate([bce, alpha], axis=1)
    x = pack.reshape(SC_PER_STEP, SUB, NCOL)
    ident = jnp.eye(SUB, dtype=jnp.float32)
    t = lax.dot_general(x, ident, (((1,), (0,)), ((), ())),
                        preferred_element_type=jnp.float32)
    out_ref[...] = t


def _make_sc_body(half):
    def _sc_body(p_h, ixp_h, out_h, acc_v, pv0, pv1, ix0, ix1,
                 sem0, sem1):
        pvs = (pv0, pv1)
        ixs = (ix0, ix1)
        sms = (sem0, sem1)
        cid = lax.axis_index("c")
        sid = lax.axis_index("s")
        wid = sid * NC + cid
        base = wid * EW

        zero16 = jnp.zeros((16,), jnp.float32)

        def zero_body(i, _):
            acc_v[pl.ds(i * 16, 16)] = zero16
            return _

        lax.fori_loop(0, NACC * B // 16, zero_body, None)

        iota = lax.iota(jnp.int32, 16)
        lane0 = iota == 0
        lane15 = iota == 15
        cnt_end = (iota + 1).astype(jnp.float32)
        cnt_start = iota.astype(jnp.float32)

        def issue(ci, slot):
            e0 = base + ci * CH
            sub0 = e0 // SUB
            pltpu.async_copy(p_h.at[pl.ds(sub0, CHS)], pvs[slot],
                             sms[slot])
            pltpu.async_copy(ixp_h.at[pl.ds(half * EH + e0, CH + 16)],
                             ixs[slot], sms[slot])

        def wait(ci, slot):
            e0 = base + ci * CH
            sub0 = e0 // SUB
            pltpu.make_async_copy(p_h.at[pl.ds(sub0, CHS)], pvs[slot],
                                  sms[slot]).wait()
            pltpu.make_async_copy(
                ixp_h.at[pl.ds(half * EH + e0, CH + 16)],
                ixs[slot], sms[slot]).wait()

        def process(ci, slot):
            pv = pvs[slot]
            ix_v = ixs[slot]
            @plsc.parallel_loop(0, NG)
            def group_body(g):
                o = g * 16
                s = g // 8
                l = g % 8
                d = ix_v[pl.ds(8 + o, 16)]
                dn = ix_v[pl.ds(9 + o, 16)]
                dp = ix_v[pl.ds(7 + o, 16)]
                m_end = (d != dn) | lane15
                m_start = (d != dp) | lane0
                lo = l * 16

                for col in range(NCOL):
                    v = pv[s, col, pl.ds(lo, 16)]
                    c_in = plsc.cumsum(v)
                    x_ex = v - c_in
                    dk = d + (col * B)
                    plsc.addupdate_scatter(acc_v, [dk], c_in, mask=m_end)
                    plsc.addupdate_scatter(acc_v, [dk], x_ex, mask=m_start)

                dc = d + (NCOL * B)
                plsc.addupdate_scatter(acc_v, [dc], cnt_end, mask=m_end)
                plsc.addupdate_scatter(acc_v, [dc], -cnt_start,
                                       mask=m_start)

        issue(0, 0)

        def outer_body(cc, _):
            for b in range(2):
                ci = cc * 2 + b

                @pl.when(ci + 1 < NCH)
                def _():
                    issue(ci + 1, (b + 1) % 2)

                wait(ci, b)
                process(ci, b)
            return _

        lax.fori_loop(0, NCH // 2, outer_body, None)
        pltpu.sync_copy(acc_v, out_h.at[wid])

    return _sc_body


def _tc_combine_kernel(p_ref, out_ref, acc_ref):
    step = pl.program_id(0)
    contrib = jnp.sum(p_ref[...], axis=(0, 1))

    @pl.when(step == 0)
    def _():
        acc_ref[...] = contrib

    @pl.when(step != 0)
    def _():
        acc_ref[...] += contrib

    @pl.when(step != NHALF - 1)
    def _():
        out_ref[...] = jnp.zeros((1, 1), jnp.float32)

    @pl.when(step == NHALF - 1)
    def _():
        _combine_epilogue(acc_ref, out_ref)


def _combine_epilogue(acc_ref, out_ref):
    S = acc_ref[...]
    nll = S[0:K]
    A = S[K:2 * K]
    n = S[2 * K:2 * K + 1]
    ra = A / n
    ra_max = jnp.max(ra, axis=0, keepdims=True)
    ls = ra - ra_max - jnp.log(
        jnp.sum(jnp.exp(ra - ra_max), axis=0, keepdims=True))
    x = -nll + ls
    x_max = jnp.max(x, axis=0, keepdims=True)
    lp = x_max + jnp.log(jnp.sum(jnp.exp(x - x_max), axis=0,
                                 keepdims=True))
    loss_b = -lp / n
    out_ref[...] = jnp.sum(loss_b, axis=1, keepdims=True) / B


def _prepass(half, label, log_theta, log_alpha, idx, idxn):
    off = half * (EH // EC2)
    return pl.pallas_call(
        _pack_kernel,
        grid=(NSTEP_H,),
        in_specs=[
            pl.BlockSpec((EC2,), lambda i: (i + off,)),
            pl.BlockSpec((EC2, K), lambda i: (i + off, 0)),
            pl.BlockSpec((EC2, K), lambda i: (i + off, 0)),
            pl.BlockSpec((EC2,), lambda i: (i + off,)),
            pl.BlockSpec((EC2,), lambda i: (i + off,)),
        ],
        out_specs=pl.BlockSpec((SC_PER_STEP, NCOL, SUB),
                               lambda i: (i, 0, 0)),
        out_shape=jax.ShapeDtypeStruct((EH // SUB, NCOL, SUB),
                                       jnp.float32),
    )(label, log_theta, log_alpha, idx, idxn)


def _sc_reduce(half, packed, ixp):
    mesh = plsc.VectorSubcoreMesh(core_axis_name="c", subcore_axis_name="s",
                                  num_cores=NC, num_subcores=NS)
    return pl.kernel(
        _make_sc_body(half),
        out_type=jax.ShapeDtypeStruct((NW, NACC * B), jnp.float32),
        mesh=mesh,
        compiler_params=pltpu.CompilerParams(needs_layout_passes=False),
        scratch_types=[
            pltpu.VMEM((NACC * B,), jnp.float32),
            pltpu.VMEM((CHS, NCOL, SUB), jnp.float32),
            pltpu.VMEM((CHS, NCOL, SUB), jnp.float32),
            pltpu.VMEM((CH + 16,), jnp.int32),
            pltpu.VMEM((CH + 16,), jnp.int32),
            pltpu.SemaphoreType.DMA,
            pltpu.SemaphoreType.DMA,
        ],
    )(packed, ixp)


@jax.jit
def _run(label, log_theta, log_alpha, subgraph_idx):
    idx = subgraph_idx.astype(jnp.int32)
    idxn = jnp.concatenate([idx[1:], jnp.full((1,), B, jnp.int32)])
    ixp = jnp.concatenate([jnp.full((8,), -1, jnp.int32), idx,
                           jnp.full((8,), B, jnp.int32)])

    parts = []
    for h in range(NHALF):
        p = _prepass(h, label, log_theta, log_alpha, idx, idxn)
        parts.append(_sc_reduce(h, p, ixp).reshape(1, NW, NACC, B))
    stacked = jnp.concatenate(parts, axis=0)

    out = pl.pallas_call(
        _tc_combine_kernel,
        grid=(NHALF,),
        in_specs=[pl.BlockSpec((1, NW, NACC, B), lambda i: (i, 0, 0, 0))],
        out_specs=pl.BlockSpec((1, 1), lambda i: (0, 0)),
        out_shape=jax.ShapeDtypeStruct((1, 1), jnp.float32),
        scratch_shapes=[pltpu.VMEM((NACC, B), jnp.float32)],
    )(stacked)
    return out[0, 0]


def kernel(label, log_theta, log_alpha, subgraph_idx, subgraph_idx_base,
           num_canonical_order):
    loss = _run(label, log_theta, log_alpha, subgraph_idx)
    return loss * jnp.asarray(num_canonical_order, jnp.float32)


# speedup vs baseline: 1.0880x; 1.0007x over previous
import functools

import jax
import jax.numpy as jnp
from jax import lax
from jax.experimental import pallas as pl
from jax.experimental.pallas import tpu as pltpu
from jax.experimental.pallas import tpu_sc as plsc

E = 1048576
K = 20
B = 2048
NCOL = 2 * K
NACC = NCOL + 1
SUB = 128

NHALF = 4
EH = E // NHALF
ROWS_H = EH // SUB * NCOL

EC2 = 8192
NSTEP_H = EH // EC2
SC_PER_STEP = EC2 // SUB
ROWS_PER_STEP = SC_PER_STEP * NCOL

NC = 2
NS = 16
NW = NC * NS
EW = EH // NW
CH = 512
CHS = CH // SUB
NCH = EW // CH
NG = CH // 16


def _pack_kernel(label_ref, theta_ref, alpha_ref, idx_ref, idxn_ref,
                 out_ref):
    theta = theta_ref[...]
    alpha = alpha_ref[...]
    lab = label_ref[...].reshape(EC2, 1)
    m = (idx_ref[...] == idxn_ref[...]).astype(jnp.float32).reshape(EC2, 1)
    bce = (jnp.maximum(theta, 0.0) - theta * lab
           + jnp.log1p(jnp.exp(-jnp.abs(theta)))) * m
    pack = jnp.concatenate([bce, alpha], axis=1)
    x = pack.reshape(SC_PER_STEP, SUB, NCOL)
    ident = jnp.eye(SUB, dtype=jnp.float32)
    t = lax.dot_general(x, ident, (((1,), (0,)), ((), ())),
                        preferred_element_type=jnp.float32)
    out_ref[...] = t


def _make_sc_body(half):
    def _sc_body(p_h, ixp_h, out_h, acc_v, pv0, pv1, ix0, ix1,
                 sem0, sem1):
        pvs = (pv0, pv1)
        ixs = (ix0, ix1)
        sms = (sem0, sem1)
        cid = lax.axis_index("c")
        sid = lax.axis_index("s")
        wid = sid * NC + cid
        base = wid * EW

        zero16 = jnp.zeros((16,), jnp.float32)

        @plsc.parallel_loop(0, NACC * B // 16)
        def zero_body(i):
            acc_v[pl.ds(i * 16, 16)] = zero16

        iota = lax.iota(jnp.int32, 16)
        lane0 = iota == 0
        lane15 = iota == 15
        cnt_end = (iota + 1).astype(jnp.float32)
        cnt_start = iota.astype(jnp.float32)

        def issue(ci, slot):
            e0 = base + ci * CH
            sub0 = e0 // SUB
            pltpu.async_copy(p_h.at[pl.ds(sub0, CHS)], pvs[slot],
                             sms[slot])
            pltpu.async_copy(ixp_h.at[pl.ds(half * EH + e0, CH + 16)],
                             ixs[slot], sms[slot])

        def wait(ci, slot):
            e0 = base + ci * CH
            sub0 = e0 // SUB
            pltpu.make_async_copy(p_h.at[pl.ds(sub0, CHS)], pvs[slot],
                                  sms[slot]).wait()
            pltpu.make_async_copy(
                ixp_h.at[pl.ds(half * EH + e0, CH + 16)],
                ixs[slot], sms[slot]).wait()

        def process(ci, slot):
            pv = pvs[slot]
            ix_v = ixs[slot]
            @plsc.parallel_loop(0, NG)
            def group_body(g):
                o = g * 16
                s = g // 8
                l = g % 8
                d = ix_v[pl.ds(8 + o, 16)]
                dn = ix_v[pl.ds(9 + o, 16)]
                dp = ix_v[pl.ds(7 + o, 16)]
                m_end = (d != dn) | lane15
                m_start = (d != dp) | lane0
                lo = l * 16

                for col in range(NCOL):
                    v = pv[s, col, pl.ds(lo, 16)]
                    c_in = plsc.cumsum(v)
                    x_ex = v - c_in
                    dk = d + (col * B)
                    plsc.addupdate_scatter(acc_v, [dk], c_in, mask=m_end)
                    plsc.addupdate_scatter(acc_v, [dk], x_ex, mask=m_start)

                dc = d + (NCOL * B)
                plsc.addupdate_scatter(acc_v, [dc], cnt_end, mask=m_end)
                plsc.addupdate_scatter(acc_v, [dc], -cnt_start,
                                       mask=m_start)

        issue(0, 0)

        def outer_body(cc, _):
            for b in range(2):
                ci = cc * 2 + b

                @pl.when(ci + 1 < NCH)
                def _():
                    issue(ci + 1, (b + 1) % 2)

                wait(ci, b)
                process(ci, b)
            return _

        lax.fori_loop(0, NCH // 2, outer_body, None)
        pltpu.sync_copy(acc_v, out_h.at[wid])

    return _sc_body


def _tc_combine_kernel(p_ref, out_ref, acc_ref):
    step = pl.program_id(0)
    contrib = jnp.sum(p_ref[...], axis=(0, 1))

    @pl.when(step == 0)
    def _():
        acc_ref[...] = contrib

    @pl.when(step != 0)
    def _():
        acc_ref[...] += contrib

    @pl.when(step != NHALF - 1)
    def _():
        out_ref[...] = jnp.zeros((1, 1), jnp.float32)

    @pl.when(step == NHALF - 1)
    def _():
        _combine_epilogue(acc_ref, out_ref)


def _combine_epilogue(acc_ref, out_ref):
    S = acc_ref[...]
    nll = S[0:K]
    A = S[K:2 * K]
    n = S[2 * K:2 * K + 1]
    ra = A / n
    ra_max = jnp.max(ra, axis=0, keepdims=True)
    ls = ra - ra_max - jnp.log(
        jnp.sum(jnp.exp(ra - ra_max), axis=0, keepdims=True))
    x = -nll + ls
    x_max = jnp.max(x, axis=0, keepdims=True)
    lp = x_max + jnp.log(jnp.sum(jnp.exp(x - x_max), axis=0,
                                 keepdims=True))
    loss_b = -lp / n
    out_ref[...] = jnp.sum(loss_b, axis=1, keepdims=True) / B


def _prepass(half, label, log_theta, log_alpha, idx, idxn):
    off = half * (EH // EC2)
    return pl.pallas_call(
        _pack_kernel,
        grid=(NSTEP_H,),
        in_specs=[
            pl.BlockSpec((EC2,), lambda i: (i + off,)),
            pl.BlockSpec((EC2, K), lambda i: (i + off, 0)),
            pl.BlockSpec((EC2, K), lambda i: (i + off, 0)),
            pl.BlockSpec((EC2,), lambda i: (i + off,)),
            pl.BlockSpec((EC2,), lambda i: (i + off,)),
        ],
        out_specs=pl.BlockSpec((SC_PER_STEP, NCOL, SUB),
                               lambda i: (i, 0, 0)),
        out_shape=jax.ShapeDtypeStruct((EH // SUB, NCOL, SUB),
                                       jnp.float32),
    )(label, log_theta, log_alpha, idx, idxn)


def _sc_reduce(half, packed, ixp):
    mesh = plsc.VectorSubcoreMesh(core_axis_name="c", subcore_axis_name="s",
                                  num_cores=NC, num_subcores=NS)
    return pl.kernel(
        _make_sc_body(half),
        out_type=jax.ShapeDtypeStruct((NW, NACC * B), jnp.float32),
        mesh=mesh,
        compiler_params=pltpu.CompilerParams(needs_layout_passes=False),
        scratch_types=[
            pltpu.VMEM((NACC * B,), jnp.float32),
            pltpu.VMEM((CHS, NCOL, SUB), jnp.float32),
            pltpu.VMEM((CHS, NCOL, SUB), jnp.float32),
            pltpu.VMEM((CH + 16,), jnp.int32),
            pltpu.VMEM((CH + 16,), jnp.int32),
            pltpu.SemaphoreType.DMA,
            pltpu.SemaphoreType.DMA,
        ],
    )(packed, ixp)


@jax.jit
def _run(label, log_theta, log_alpha, subgraph_idx):
    idx = subgraph_idx.astype(jnp.int32)
    idxn = jnp.concatenate([idx[1:], jnp.full((1,), B, jnp.int32)])
    ixp = jnp.concatenate([jnp.full((8,), -1, jnp.int32), idx,
                           jnp.full((8,), B, jnp.int32)])

    parts = []
    for h in range(NHALF):
        p = _prepass(h, label, log_theta, log_alpha, idx, idxn)
        parts.append(_sc_reduce(h, p, ixp).reshape(1, NW, NACC, B))
    stacked = jnp.concatenate(parts, axis=0)

    out = pl.pallas_call(
        _tc_combine_kernel,
        grid=(NHALF,),
        in_specs=[pl.BlockSpec((1, NW, NACC, B), lambda i: (i, 0, 0, 0))],
        out_specs=pl.BlockSpec((1, 1), lambda i: (0, 0)),
        out_shape=jax.ShapeDtypeStruct((1, 1), jnp.float32),
        scratch_shapes=[pltpu.VMEM((NACC, B), jnp.float32)],
    )(stacked)
    return out[0, 0]


def kernel(label, log_theta, log_alpha, subgraph_idx, subgraph_idx_base,
           num_canonical_order):
    loss = _run(label, log_theta, log_alpha, subgraph_idx)
    return loss * jnp.asarray(num_canonical_order, jnp.float32)
